# Initial kernel scaffold; baseline (speedup 1.0000x reference)
import jax
import jax.numpy as jnp
from jax.experimental import pallas as pl


def kernel(x, edge_index, edge_attr, W1, b1, W2, b2, Wf1, bf1, Wf2, bf2):
    raise NotImplementedError("write your pallas kernel here")


# speedup vs baseline: 27.5852x; 27.5852x over previous
import functools

import jax
import jax.numpy as jnp
from jax import lax
from jax.experimental import pallas as pl
from jax.experimental.pallas import tpu as pltpu
from jax.experimental.pallas import tpu_sc as plsc

f32 = jnp.float32
i32 = jnp.int32

N = 100000
E = 1600000
NC, NS = 2, 16
N_ACC = 100352
ROWS = 12544
E_PAD = ROWS * 128
GR = 8
BLK = 1024
GRID = N_ACC // BLK

_MESH = dict(core_axis_name="c", subcore_axis_name="s", num_cores=NC,
             num_subcores=NS)
_SC_PARAMS = pltpu.CompilerParams(use_tc_tiling_on_sc=False)


def _make_deg():
    rpw = ROWS // (NC * NS)
    ngroups = rpw // GR
    srows = N_ACC // NS

    @functools.partial(
        pl.kernel,
        out_type=jax.ShapeDtypeStruct((NC, N_ACC, 8), f32),
        mesh=plsc.VectorSubcoreMesh(**_MESH),
        compiler_params=_SC_PARAMS,
        scratch_types=[
            pltpu.VMEM((GR, 128), i32),
            pltpu.VMEM((128, 8), f32),
            pltpu.VMEM_SHARED((N_ACC, 8), f32),
        ],
    )
    def deg_kernel(dst_hbm, zeros_hbm, ones_hbm, out_hbm, idx_v, ones_v,
                   acc_sh):
        c = lax.axis_index("c")
        s = lax.axis_index("s")
        w = c * NS + s
        pltpu.sync_copy(ones_hbm, ones_v)
        pltpu.sync_copy(zeros_hbm.at[pl.ds(s * srows, srows)],
                        acc_sh.at[pl.ds(s * srows, srows)])
        plsc.subcore_barrier()

        def body(g, carry):
            base = w * rpw + g * GR
            pltpu.sync_copy(dst_hbm.at[pl.ds(base, GR)], idx_v)
            for j in range(GR):
                pltpu.sync_copy(ones_v, acc_sh.at[idx_v.at[j]], add=True)
            return carry

        lax.fori_loop(0, ngroups, body, 0)
        plsc.subcore_barrier()
        pltpu.sync_copy(acc_sh.at[pl.ds(s * srows, srows)],
                        out_hbm.at[c, pl.ds(s * srows, srows)])

    return deg_kernel


def _make_agg(width, split_by_worker):
    rpw = ROWS // (NC * NS) if split_by_worker else ROWS // NS
    ngroups = rpw // GR
    srows = N_ACC // NS

    @functools.partial(
        pl.kernel,
        out_type=jax.ShapeDtypeStruct((NC, N_ACC, width), f32),
        mesh=plsc.VectorSubcoreMesh(**_MESH),
        compiler_params=_SC_PARAMS,
        scratch_types=[
            pltpu.VMEM((GR, 128), i32),
            pltpu.VMEM((GR, 128), i32),
            pltpu.VMEM((GR * 128, width), f32),
            pltpu.VMEM_SHARED((N_ACC, width), f32),
            pltpu.SemaphoreType.DMA,
        ],
    )
    def agg_kernel(table_hbm, src_hbm, dst_hbm, zeros_hbm, out_hbm,
                   sidx_v, didx_v, rows_v, acc_sh, sem):
        c = lax.axis_index("c")
        s = lax.axis_index("s")
        if split_by_worker:
            base0 = (c * NS + s) * rpw
            tbl = table_hbm
        else:
            base0 = s * rpw
            tbl = table_hbm.at[c]
        pltpu.sync_copy(zeros_hbm.at[pl.ds(s * srows, srows)],
                        acc_sh.at[pl.ds(s * srows, srows)])
        plsc.subcore_barrier()

        def body(g, carry):
            base = base0 + g * GR
            pltpu.sync_copy(src_hbm.at[pl.ds(base, GR)], sidx_v)
            pltpu.sync_copy(dst_hbm.at[pl.ds(base, GR)], didx_v)
            descs = [
                pltpu.async_copy(tbl.at[sidx_v.at[j]],
                                 rows_v.at[pl.ds(j * 128, 128)], sem)
                for j in range(GR)
            ]
            for d in descs:
                d.wait()
            for j in range(GR):
                pltpu.sync_copy(rows_v.at[pl.ds(j * 128, 128)],
                                acc_sh.at[didx_v.at[j]], add=True)
            return carry

        lax.fori_loop(0, ngroups, body, 0)
        plsc.subcore_barrier()
        pltpu.sync_copy(acc_sh.at[pl.ds(s * srows, srows)],
                        out_hbm.at[c, pl.ds(s * srows, srows)])

    return agg_kernel


_deg = _make_deg()
_p1 = _make_agg(8, True)
_p2 = _make_agg(16, False)


def _d1_body(degp_ref, ea_ref, dinv_ref, y1_ref):
    deg = degp_ref[0, :, 0:1] + degp_ref[1, :, 0:1] + 1.0
    dinv = 1.0 / jnp.sqrt(deg)
    dinv_ref[...] = dinv
    y1_ref[...] = ea_ref[...] * dinv


_d1 = pl.pallas_call(
    _d1_body,
    grid=(GRID,),
    in_specs=[
        pl.BlockSpec((NC, BLK, 8), lambda i: (0, i, 0)),
        pl.BlockSpec((BLK, 8), lambda i: (i, 0)),
    ],
    out_specs=[
        pl.BlockSpec((BLK, 1), lambda i: (i, 0)),
        pl.BlockSpec((BLK, 8), lambda i: (i, 0)),
    ],
    out_shape=[
        jax.ShapeDtypeStruct((N_ACC, 1), f32),
        jax.ShapeDtypeStruct((N_ACC, 8), f32),
    ],
)


def _d2_body(z1_ref, y1_ref, dinv_ref, w1_ref, b1_ref, w2_ref, y2_ref):
    dinv = dinv_ref[...]
    agg1 = (z1_ref[0] + z1_ref[1] + y1_ref[...]) * dinv
    h1 = jnp.maximum(
        jnp.dot(agg1, w1_ref[...], preferred_element_type=f32) + b1_ref[...],
        0.0)
    y2 = jnp.dot(h1, w2_ref[...], preferred_element_type=f32) * dinv
    y2_ref[0] = y2[:, :16]
    y2_ref[1] = y2[:, 16:32]


_d2 = pl.pallas_call(
    _d2_body,
    grid=(GRID,),
    in_specs=[
        pl.BlockSpec((NC, BLK, 8), lambda i: (0, i, 0)),
        pl.BlockSpec((BLK, 8), lambda i: (i, 0)),
        pl.BlockSpec((BLK, 1), lambda i: (i, 0)),
        pl.BlockSpec((8, 128), lambda i: (0, 0)),
        pl.BlockSpec((1, 128), lambda i: (0, 0)),
        pl.BlockSpec((128, 128), lambda i: (0, 0)),
    ],
    out_specs=pl.BlockSpec((NC, BLK, 16), lambda i: (0, i, 0)),
    out_shape=jax.ShapeDtypeStruct((NC, N_ACC, 16), f32),
)


def _d3_body(z2_ref, y2_ref, dinv_ref, b2_ref, wf1_ref, bf1_ref, wf2_ref,
             bf2_ref, o_ref):
    dinv = dinv_ref[...]
    u = jnp.concatenate(
        [z2_ref[0] + y2_ref[0], z2_ref[1] + y2_ref[1]], axis=1) * dinv
    up = jnp.pad(u, ((0, 0), (0, 96)))
    h2 = jnp.maximum(up + b2_ref[...], 0.0)
    h3 = jnp.maximum(
        jnp.dot(h2, wf1_ref[...], preferred_element_type=f32) + bf1_ref[...],
        0.0)
    o = jnp.dot(h3, wf2_ref[...], preferred_element_type=f32) + bf2_ref[...]
    o_ref[...] = o[:, :2]


_d3 = pl.pallas_call(
    _d3_body,
    grid=(GRID,),
    in_specs=[
        pl.BlockSpec((NC, BLK, 16), lambda i: (0, i, 0)),
        pl.BlockSpec((NC, BLK, 16), lambda i: (0, i, 0)),
        pl.BlockSpec((BLK, 1), lambda i: (i, 0)),
        pl.BlockSpec((1, 128), lambda i: (0, 0)),
        pl.BlockSpec((128, 128), lambda i: (0, 0)),
        pl.BlockSpec((1, 128), lambda i: (0, 0)),
        pl.BlockSpec((128, 128), lambda i: (0, 0)),
        pl.BlockSpec((1, 128), lambda i: (0, 0)),
    ],
    out_specs=pl.BlockSpec((BLK, 2), lambda i: (i, 0)),
    out_shape=jax.ShapeDtypeStruct((N_ACC, 2), f32),
)


def kernel(x, edge_index, edge_attr, W1, b1, W2, b2, Wf1, bf1, Wf2, bf2):
    ei = edge_index.astype(i32)
    pad = jnp.full((E_PAD - E,), N, i32)
    src_rows = jnp.concatenate([ei[0], pad]).reshape(ROWS, 128)
    dst_rows = jnp.concatenate([ei[1], pad]).reshape(ROWS, 128)
    ea_pad = jnp.pad(edge_attr, ((0, N_ACC - N), (0, 3)))
    W1p = jnp.pad(W1, ((0, 3), (0, 64)))
    b1p = jnp.pad(b1, (0, 64)).reshape(1, 128)
    W2p = jnp.pad(W2, ((0, 64), (0, 96)))
    b2p = jnp.pad(b2, (0, 96)).reshape(1, 128)
    Wf1p = jnp.pad(Wf1, ((0, 96), (0, 112)))
    bf1p = jnp.pad(bf1, (0, 112)).reshape(1, 128)
    Wf2p = jnp.pad(Wf2, ((0, 112), (0, 126)))
    bf2p = jnp.pad(bf2, (0, 126)).reshape(1, 128)
    zeros8 = jnp.zeros((N_ACC, 8), f32)
    zeros16 = jnp.zeros((N_ACC, 16), f32)
    ones8 = jnp.zeros((128, 8), f32).at[:, 0].set(1.0)

    degp = _deg(dst_rows, zeros8, ones8)
    dinv, y1 = _d1(degp, ea_pad)
    z1 = _p1(y1, src_rows, dst_rows, zeros8)
    y2 = _d2(z1, y1, dinv, W1p, b1p, W2p)
    z2 = _p2(y2, src_rows, dst_rows, zeros16)
    o = _d3(z2, y2, dinv, b2p, Wf1p, bf1p, Wf2p, bf2p)
    return o[:N]


# speedup vs baseline: 33.7893x; 1.2249x over previous
import functools

import jax
import jax.numpy as jnp
from jax import lax
from jax.experimental import pallas as pl
from jax.experimental.pallas import tpu as pltpu
from jax.experimental.pallas import tpu_sc as plsc

f32 = jnp.float32
i32 = jnp.int32

N = 100000
E = 1600000
NC, NS = 2, 16
N_ACC = 100352
ROWS = 12544
E_PAD = ROWS * 128
GR = 4
BLK = 1024
GRID = N_ACC // BLK

_MESH = dict(core_axis_name="c", subcore_axis_name="s", num_cores=NC,
             num_subcores=NS)
_SC_PARAMS = pltpu.CompilerParams(use_tc_tiling_on_sc=False)


def _make_deg():
    rpw = ROWS // (NC * NS)
    ngroups = rpw // GR
    npairs = ngroups // 2
    srows = N_ACC // NS

    @functools.partial(
        pl.kernel,
        out_type=jax.ShapeDtypeStruct((NC, N_ACC, 8), f32),
        mesh=plsc.VectorSubcoreMesh(**_MESH),
        compiler_params=_SC_PARAMS,
        scratch_types=[
            pltpu.VMEM((GR, 128), i32),
            pltpu.VMEM((GR, 128), i32),
            pltpu.VMEM((128, 8), f32),
            pltpu.VMEM_SHARED((N_ACC, 8), f32),
            pltpu.SemaphoreType.DMA,
            pltpu.SemaphoreType.DMA,
        ],
    )
    def deg_kernel(dst_hbm, zeros_hbm, ones_hbm, out_hbm, idx0, idx1,
                   ones_v, acc_sh, isem0, isem1):
        c = lax.axis_index("c")
        s = lax.axis_index("s")
        w = c * NS + s
        base0 = w * rpw

        def iload(g, buf, sem):
            pltpu.async_copy(dst_hbm.at[pl.ds(base0 + g * GR, GR)], buf, sem)

        def iwait(buf, sem):
            pltpu.make_async_copy(dst_hbm.at[pl.ds(base0, GR)], buf,
                                  sem).wait()

        def scatters(buf):
            for j in range(GR):
                pltpu.sync_copy(ones_v, acc_sh.at[buf.at[j]], add=True)

        iload(0, idx0, isem0)
        iload(1, idx1, isem1)
        pltpu.sync_copy(ones_hbm, ones_v)
        pltpu.sync_copy(zeros_hbm.at[pl.ds(s * srows, srows)],
                        acc_sh.at[pl.ds(s * srows, srows)])
        plsc.subcore_barrier()

        def body(k, carry):
            g = 2 * k
            iwait(idx0, isem0)
            scatters(idx0)

            @pl.when(g + 2 < ngroups)
            def _():
                iload(g + 2, idx0, isem0)

            iwait(idx1, isem1)
            scatters(idx1)

            @pl.when(g + 3 < ngroups)
            def _():
                iload(g + 3, idx1, isem1)

            return carry

        lax.fori_loop(0, npairs, body, 0)
        plsc.subcore_barrier()
        pltpu.sync_copy(acc_sh.at[pl.ds(s * srows, srows)],
                        out_hbm.at[c, pl.ds(s * srows, srows)])

    return deg_kernel


def _make_agg(width, split_by_worker):
    rpw = ROWS // (NC * NS) if split_by_worker else ROWS // NS
    ngroups = rpw // GR
    npairs = ngroups // 2
    srows = N_ACC // NS

    @functools.partial(
        pl.kernel,
        out_type=jax.ShapeDtypeStruct((NC, N_ACC, width), f32),
        mesh=plsc.VectorSubcoreMesh(**_MESH),
        compiler_params=_SC_PARAMS,
        scratch_types=[
            pltpu.VMEM((GR, 128), i32),
            pltpu.VMEM((GR, 128), i32),
            pltpu.VMEM((GR, 128), i32),
            pltpu.VMEM((GR, 128), i32),
            pltpu.VMEM((GR * 128, width), f32),
            pltpu.VMEM((GR * 128, width), f32),
            pltpu.VMEM_SHARED((N_ACC, width), f32),
            pltpu.SemaphoreType.DMA,
            pltpu.SemaphoreType.DMA,
            pltpu.SemaphoreType.DMA,
            pltpu.SemaphoreType.DMA,
        ],
    )
    def agg_kernel(table_hbm, src_hbm, dst_hbm, zeros_hbm, out_hbm,
                   sidx0, sidx1, didx0, didx1, rows0, rows1, acc_sh,
                   isem0, isem1, gsem0, gsem1):
        c = lax.axis_index("c")
        s = lax.axis_index("s")
        if split_by_worker:
            base0 = (c * NS + s) * rpw
            tbl = table_hbm
        else:
            base0 = s * rpw
            tbl = table_hbm.at[c]

        def sload(g, sbuf, sem):
            pltpu.async_copy(src_hbm.at[pl.ds(base0 + g * GR, GR)], sbuf, sem)

        def dload(g, dbuf, sem):
            pltpu.async_copy(dst_hbm.at[pl.ds(base0 + g * GR, GR)], dbuf, sem)

        def ibwait(buf, sem):
            pltpu.make_async_copy(src_hbm.at[pl.ds(base0, GR)], buf,
                                  sem).wait()

        def gathers(sbuf, rbuf, sem):
            for j in range(GR):
                pltpu.async_copy(tbl.at[sbuf.at[j]],
                                 rbuf.at[pl.ds(j * 128, 128)], sem)

        def gwait(rbuf, sem):
            for j in range(GR):
                pltpu.make_async_copy(tbl.at[pl.ds(0, 128)],
                                      rbuf.at[pl.ds(j * 128, 128)],
                                      sem).wait()

        def scatters(rbuf, dbuf):
            for j in range(GR):
                pltpu.sync_copy(rbuf.at[pl.ds(j * 128, 128)],
                                acc_sh.at[dbuf.at[j]], add=True)

        sload(0, sidx0, isem0)
        dload(0, didx0, isem0)
        sload(1, sidx1, isem1)
        dload(1, didx1, isem1)
        ibwait(sidx0, isem0)
        ibwait(didx0, isem0)
        gathers(sidx0, rows0, gsem0)
        pltpu.sync_copy(zeros_hbm.at[pl.ds(s * srows, srows)],
                        acc_sh.at[pl.ds(s * srows, srows)])
        plsc.subcore_barrier()

        def body(k, carry):
            g = 2 * k
            ibwait(sidx1, isem1)
            ibwait(didx1, isem1)
            gathers(sidx1, rows1, gsem1)
            gwait(rows0, gsem0)

            @pl.when(g + 2 < ngroups)
            def _():
                sload(g + 2, sidx0, isem0)

            scatters(rows0, didx0)

            @pl.when(g + 2 < ngroups)
            def _():
                dload(g + 2, didx0, isem0)
                ibwait(sidx0, isem0)
                ibwait(didx0, isem0)
                gathers(sidx0, rows0, gsem0)

            gwait(rows1, gsem1)

            @pl.when(g + 3 < ngroups)
            def _():
                sload(g + 3, sidx1, isem1)

            scatters(rows1, didx1)

            @pl.when(g + 3 < ngroups)
            def _():
                dload(g + 3, didx1, isem1)

            return carry

        lax.fori_loop(0, npairs, body, 0)
        plsc.subcore_barrier()
        pltpu.sync_copy(acc_sh.at[pl.ds(s * srows, srows)],
                        out_hbm.at[c, pl.ds(s * srows, srows)])

    return agg_kernel


_deg = _make_deg()
_p1 = _make_agg(8, True)
_p2 = _make_agg(16, False)


def _d1_body(degp_ref, ea_ref, dinv_ref, y1_ref):
    deg = degp_ref[0, :, 0:1] + degp_ref[1, :, 0:1] + 1.0
    dinv = 1.0 / jnp.sqrt(deg)
    dinv_ref[...] = dinv
    y1_ref[...] = ea_ref[...] * dinv


_d1 = pl.pallas_call(
    _d1_body,
    grid=(GRID,),
    in_specs=[
        pl.BlockSpec((NC, BLK, 8), lambda i: (0, i, 0)),
        pl.BlockSpec((BLK, 8), lambda i: (i, 0)),
    ],
    out_specs=[
        pl.BlockSpec((BLK, 1), lambda i: (i, 0)),
        pl.BlockSpec((BLK, 8), lambda i: (i, 0)),
    ],
    out_shape=[
        jax.ShapeDtypeStruct((N_ACC, 1), f32),
        jax.ShapeDtypeStruct((N_ACC, 8), f32),
    ],
)


def _d2_body(z1_ref, y1_ref, dinv_ref, w1_ref, b1_ref, w2_ref, y2_ref):
    dinv = dinv_ref[...]
    agg1 = (z1_ref[0] + z1_ref[1] + y1_ref[...]) * dinv
    h1 = jnp.maximum(
        jnp.dot(agg1, w1_ref[...], preferred_element_type=f32) + b1_ref[...],
        0.0)
    y2 = jnp.dot(h1, w2_ref[...], preferred_element_type=f32) * dinv
    y2_ref[0] = y2[:, :16]
    y2_ref[1] = y2[:, 16:32]


_d2 = pl.pallas_call(
    _d2_body,
    grid=(GRID,),
    in_specs=[
        pl.BlockSpec((NC, BLK, 8), lambda i: (0, i, 0)),
        pl.BlockSpec((BLK, 8), lambda i: (i, 0)),
        pl.BlockSpec((BLK, 1), lambda i: (i, 0)),
        pl.BlockSpec((8, 128), lambda i: (0, 0)),
        pl.BlockSpec((1, 128), lambda i: (0, 0)),
        pl.BlockSpec((128, 128), lambda i: (0, 0)),
    ],
    out_specs=pl.BlockSpec((NC, BLK, 16), lambda i: (0, i, 0)),
    out_shape=jax.ShapeDtypeStruct((NC, N_ACC, 16), f32),
)


def _d3_body(z2_ref, y2_ref, dinv_ref, b2_ref, wf1_ref, bf1_ref, wf2_ref,
             bf2_ref, o_ref):
    dinv = dinv_ref[...]
    u = jnp.concatenate(
        [z2_ref[0] + y2_ref[0], z2_ref[1] + y2_ref[1]], axis=1) * dinv
    up = jnp.pad(u, ((0, 0), (0, 96)))
    h2 = jnp.maximum(up + b2_ref[...], 0.0)
    h3 = jnp.maximum(
        jnp.dot(h2, wf1_ref[...], preferred_element_type=f32) + bf1_ref[...],
        0.0)
    o = jnp.dot(h3, wf2_ref[...], preferred_element_type=f32) + bf2_ref[...]
    o_ref[...] = o[:, :2]


_d3 = pl.pallas_call(
    _d3_body,
    grid=(GRID,),
    in_specs=[
        pl.BlockSpec((NC, BLK, 16), lambda i: (0, i, 0)),
        pl.BlockSpec((NC, BLK, 16), lambda i: (0, i, 0)),
        pl.BlockSpec((BLK, 1), lambda i: (i, 0)),
        pl.BlockSpec((1, 128), lambda i: (0, 0)),
        pl.BlockSpec((128, 128), lambda i: (0, 0)),
        pl.BlockSpec((1, 128), lambda i: (0, 0)),
        pl.BlockSpec((128, 128), lambda i: (0, 0)),
        pl.BlockSpec((1, 128), lambda i: (0, 0)),
    ],
    out_specs=pl.BlockSpec((BLK, 2), lambda i: (i, 0)),
    out_shape=jax.ShapeDtypeStruct((N, 2), f32),
)


def kernel(x, edge_index, edge_attr, W1, b1, W2, b2, Wf1, bf1, Wf2, bf2):
    ei = edge_index.astype(i32)
    pad = jnp.full((E_PAD - E,), N, i32)
    src_rows = jnp.concatenate([ei[0], pad]).reshape(ROWS, 128)
    dst_rows = jnp.concatenate([ei[1], pad]).reshape(ROWS, 128)
    ea_pad = jnp.pad(edge_attr, ((0, N_ACC - N), (0, 3)))
    W1p = jnp.pad(W1, ((0, 3), (0, 64)))
    b1p = jnp.pad(b1, (0, 64)).reshape(1, 128)
    W2p = jnp.pad(W2, ((0, 64), (0, 96)))
    b2p = jnp.pad(b2, (0, 96)).reshape(1, 128)
    Wf1p = jnp.pad(Wf1, ((0, 96), (0, 112)))
    bf1p = jnp.pad(bf1, (0, 112)).reshape(1, 128)
    Wf2p = jnp.pad(Wf2, ((0, 112), (0, 126)))
    bf2p = jnp.pad(bf2, (0, 126)).reshape(1, 128)
    zeros8 = jnp.zeros((N_ACC, 8), f32)
    zeros16 = jnp.zeros((N_ACC, 16), f32)
    ones8 = jnp.zeros((128, 8), f32).at[:, 0].set(1.0)

    degp = _deg(dst_rows, zeros8, ones8)
    dinv, y1 = _d1(degp, ea_pad)
    z1 = _p1(y1, src_rows, dst_rows, zeros8)
    y2 = _d2(z1, y1, dinv, W1p, b1p, W2p)
    z2 = _p2(y2, src_rows, dst_rows, zeros16)
    return _d3(z2, y2, dinv, b2p, Wf1p, bf1p, Wf2p, bf2p)


# speedup vs baseline: 50.3180x; 1.4892x over previous
import functools

import jax
import jax.numpy as jnp
from jax import lax
from jax.experimental import pallas as pl
from jax.experimental.pallas import tpu as pltpu
from jax.experimental.pallas import tpu_sc as plsc

f32 = jnp.float32
i32 = jnp.int32

N = 100000
E = 1600000
NC, NS = 2, 16
N_ACC = 100352
ROWS = 12544
E_PAD = ROWS * 128
GR = 4
BLK = 1024
GRID = N_ACC // BLK

_MESH = dict(core_axis_name="c", subcore_axis_name="s", num_cores=NC,
             num_subcores=NS)
_SC_PARAMS = pltpu.CompilerParams(use_tc_tiling_on_sc=False)


def _make_deg():
    rpw = ROWS // (NC * NS)
    ngroups = rpw // GR
    npairs = ngroups // 2
    srows = N_ACC // NS

    @functools.partial(
        pl.kernel,
        out_type=jax.ShapeDtypeStruct((NC, N_ACC, 8), f32),
        mesh=plsc.VectorSubcoreMesh(**_MESH),
        compiler_params=_SC_PARAMS,
        scratch_types=[
            pltpu.VMEM((GR, 128), i32),
            pltpu.VMEM((GR, 128), i32),
            pltpu.VMEM((128, 8), f32),
            pltpu.VMEM_SHARED((N_ACC, 8), f32),
            pltpu.SemaphoreType.DMA,
            pltpu.SemaphoreType.DMA,
        ],
    )
    def deg_kernel(dst_hbm, zeros_hbm, ones_hbm, out_hbm, idx0, idx1,
                   ones_v, acc_sh, isem0, isem1):
        c = lax.axis_index("c")
        s = lax.axis_index("s")
        w = c * NS + s
        base0 = w * rpw

        def iload(g, buf, sem):
            pltpu.async_copy(dst_hbm.at[pl.ds(base0 + g * GR, GR)], buf, sem)

        def iwait(buf, sem):
            pltpu.make_async_copy(dst_hbm.at[pl.ds(base0, GR)], buf,
                                  sem).wait()

        def scatters(buf):
            for j in range(GR):
                pltpu.sync_copy(ones_v, acc_sh.at[buf.at[j]], add=True)

        iload(0, idx0, isem0)
        iload(1, idx1, isem1)
        pltpu.sync_copy(ones_hbm, ones_v)
        pltpu.sync_copy(zeros_hbm.at[pl.ds(s * srows, srows)],
                        acc_sh.at[pl.ds(s * srows, srows)])
        plsc.subcore_barrier()

        def body(k, carry):
            g = 2 * k
            iwait(idx0, isem0)
            scatters(idx0)

            @pl.when(g + 2 < ngroups)
            def _():
                iload(g + 2, idx0, isem0)

            iwait(idx1, isem1)
            scatters(idx1)

            @pl.when(g + 3 < ngroups)
            def _():
                iload(g + 3, idx1, isem1)

            return carry

        lax.fori_loop(0, npairs, body, 0)
        plsc.subcore_barrier()
        pltpu.sync_copy(acc_sh.at[pl.ds(s * srows, srows)],
                        out_hbm.at[c, pl.ds(s * srows, srows)])

    return deg_kernel


def _make_agg(width, split_by_worker):
    rpw = ROWS // (NC * NS) if split_by_worker else ROWS // NS
    ngroups = rpw // GR
    npairs = ngroups // 2
    srows = N_ACC // NS
    out_shape = ((NC, N_ACC, width) if split_by_worker
                 else (N_ACC, NC, width))

    @functools.partial(
        pl.kernel,
        out_type=jax.ShapeDtypeStruct(out_shape, f32),
        mesh=plsc.VectorSubcoreMesh(**_MESH),
        compiler_params=_SC_PARAMS,
        scratch_types=[
            pltpu.VMEM((GR, 128), i32),
            pltpu.VMEM((GR, 128), i32),
            pltpu.VMEM((GR, 128), i32),
            pltpu.VMEM((GR, 128), i32),
            pltpu.VMEM((GR * 128, width), f32),
            pltpu.VMEM((GR * 128, width), f32),
            pltpu.VMEM_SHARED((N_ACC, width), f32),
            pltpu.SemaphoreType.DMA,
            pltpu.SemaphoreType.DMA,
            pltpu.SemaphoreType.DMA,
            pltpu.SemaphoreType.DMA,
        ],
    )
    def agg_kernel(table_hbm, src_hbm, dst_hbm, zeros_hbm, out_hbm,
                   sidx0, sidx1, didx0, didx1, rows0, rows1, acc_sh,
                   isem0, isem1, gsem0, gsem1):
        c = lax.axis_index("c")
        s = lax.axis_index("s")
        if split_by_worker:
            base0 = (c * NS + s) * rpw
            tbl = table_hbm
            sref = src_hbm
        else:
            base0 = s * rpw
            tbl = table_hbm
            sref = src_hbm.at[c]

        def sload(g, sbuf, sem):
            pltpu.async_copy(sref.at[pl.ds(base0 + g * GR, GR)], sbuf, sem)

        def dload(g, dbuf, sem):
            pltpu.async_copy(dst_hbm.at[pl.ds(base0 + g * GR, GR)], dbuf, sem)

        def ibwait(buf, sem):
            pltpu.make_async_copy(dst_hbm.at[pl.ds(base0, GR)], buf,
                                  sem).wait()

        def gathers(sbuf, rbuf, sem):
            for j in range(GR):
                pltpu.async_copy(tbl.at[sbuf.at[j]],
                                 rbuf.at[pl.ds(j * 128, 128)], sem)

        def gwait(rbuf, sem):
            for j in range(GR):
                pltpu.make_async_copy(tbl.at[pl.ds(0, 128)],
                                      rbuf.at[pl.ds(j * 128, 128)],
                                      sem).wait()

        def scatters(rbuf, dbuf):
            for j in range(GR):
                pltpu.sync_copy(rbuf.at[pl.ds(j * 128, 128)],
                                acc_sh.at[dbuf.at[j]], add=True)

        sload(0, sidx0, isem0)
        dload(0, didx0, isem0)
        sload(1, sidx1, isem1)
        dload(1, didx1, isem1)
        ibwait(sidx0, isem0)
        ibwait(didx0, isem0)
        gathers(sidx0, rows0, gsem0)
        pltpu.sync_copy(zeros_hbm.at[pl.ds(s * srows, srows)],
                        acc_sh.at[pl.ds(s * srows, srows)])
        plsc.subcore_barrier()

        def body(k, carry):
            g = 2 * k
            ibwait(sidx1, isem1)
            ibwait(didx1, isem1)
            gathers(sidx1, rows1, gsem1)
            gwait(rows0, gsem0)

            @pl.when(g + 2 < ngroups)
            def _():
                sload(g + 2, sidx0, isem0)

            scatters(rows0, didx0)

            @pl.when(g + 2 < ngroups)
            def _():
                dload(g + 2, didx0, isem0)
                ibwait(sidx0, isem0)
                ibwait(didx0, isem0)
                gathers(sidx0, rows0, gsem0)

            gwait(rows1, gsem1)

            @pl.when(g + 3 < ngroups)
            def _():
                sload(g + 3, sidx1, isem1)

            scatters(rows1, didx1)

            @pl.when(g + 3 < ngroups)
            def _():
                dload(g + 3, didx1, isem1)

            return carry

        lax.fori_loop(0, npairs, body, 0)
        plsc.subcore_barrier()
        if split_by_worker:
            pltpu.sync_copy(acc_sh.at[pl.ds(s * srows, srows)],
                            out_hbm.at[c, pl.ds(s * srows, srows)])
        else:
            pltpu.sync_copy(acc_sh.at[pl.ds(s * srows, srows)],
                            out_hbm.at[pl.ds(s * srows, srows), c])

    return agg_kernel


_deg = _make_deg()
_p1 = _make_agg(8, True)
_p2 = _make_agg(16, False)

BLKP = 128
GRIDP = (N_ACC // 16) // BLKP
BLKQ = 512
GRIDQ = (N_ACC // 4) // BLKQ


def _d1_body(degp_ref, eap_ref, s8_ref, e8_ref, dinv_ref, y1p_ref):
    degsum = degp_ref[0] + degp_ref[1]
    deg = jnp.dot(degsum, s8_ref[...], preferred_element_type=f32) + 1.0
    dinv = 1.0 / jnp.sqrt(deg)
    dinv_ref[...] = dinv
    y1p_ref[...] = eap_ref[...] * jnp.dot(dinv, e8_ref[...],
                                          preferred_element_type=f32)


_d1 = pl.pallas_call(
    _d1_body,
    grid=(GRIDP,),
    in_specs=[
        pl.BlockSpec((NC, BLKP, 128), lambda i: (0, i, 0)),
        pl.BlockSpec((BLKP, 128), lambda i: (i, 0)),
        pl.BlockSpec((128, 16), lambda i: (0, 0)),
        pl.BlockSpec((16, 128), lambda i: (0, 0)),
    ],
    out_specs=[
        pl.BlockSpec((BLKP, 16), lambda i: (i, 0)),
        pl.BlockSpec((BLKP, 128), lambda i: (i, 0)),
    ],
    out_shape=[
        jax.ShapeDtypeStruct((N_ACC // 16, 16), f32),
        jax.ShapeDtypeStruct((N_ACC // 16, 128), f32),
    ],
)


def _d2_body(z1p_ref, y1p_ref, dinv_ref, e8_ref, w1x_ref, b1_ref, w2_ref,
             y2q_ref, dinvq_ref):
    dinv = dinv_ref[...]
    agg1p = (z1p_ref[0] + z1p_ref[1] + y1p_ref[...]) * jnp.dot(
        dinv, e8_ref[...], preferred_element_type=f32)
    for q in range(4):
        parts = []
        dparts = []
        for j in range(4):
            i = 4 * q + j
            h = jnp.maximum(
                jnp.dot(agg1p, w1x_ref[i], preferred_element_type=f32)
                + b1_ref[...], 0.0)
            t = jnp.dot(h, w2_ref[...], preferred_element_type=f32)
            db = dinv[:, i:i + 1]
            parts.append((t * db)[:, :32])
            dparts.append(jnp.broadcast_to(db, (BLKP, 32)))
        y2q_ref[q] = jnp.concatenate(parts, axis=1)
        dinvq_ref[q] = jnp.concatenate(dparts, axis=1)


_d2 = pl.pallas_call(
    _d2_body,
    grid=(GRIDP,),
    in_specs=[
        pl.BlockSpec((NC, BLKP, 128), lambda i: (0, i, 0)),
        pl.BlockSpec((BLKP, 128), lambda i: (i, 0)),
        pl.BlockSpec((BLKP, 16), lambda i: (i, 0)),
        pl.BlockSpec((16, 128), lambda i: (0, 0)),
        pl.BlockSpec((16, 128, 128), lambda i: (0, 0, 0)),
        pl.BlockSpec((1, 128), lambda i: (0, 0)),
        pl.BlockSpec((128, 128), lambda i: (0, 0)),
    ],
    out_specs=[
        pl.BlockSpec((4, BLKP, 128), lambda i: (0, i, 0)),
        pl.BlockSpec((4, BLKP, 128), lambda i: (0, i, 0)),
    ],
    out_shape=[
        jax.ShapeDtypeStruct((4, N_ACC // 16, 128), f32),
        jax.ShapeDtypeStruct((4, N_ACC // 16, 128), f32),
    ],
)


def _d3_body(z2p_ref, y2q_ref, dinvq_ref, b2_ref, wf1x_ref, bf1_ref,
             wf2_ref, bf2_ref, oq_ref):
    u = dinvq_ref[...] * (z2p_ref[...] + y2q_ref[...])
    h2 = jnp.maximum(u + b2_ref[...], 0.0)
    parts = []
    for j in range(4):
        h3 = jnp.maximum(
            jnp.dot(h2, wf1x_ref[j], preferred_element_type=f32)
            + bf1_ref[...], 0.0)
        o = jnp.dot(h3, wf2_ref[...], preferred_element_type=f32) \
            + bf2_ref[...]
        parts.append(o[:, :2])
    oq_ref[...] = jnp.concatenate(parts, axis=1)


_d3 = pl.pallas_call(
    _d3_body,
    grid=(GRIDQ,),
    in_specs=[
        pl.BlockSpec((BLKQ, 128), lambda i: (i, 0)),
        pl.BlockSpec((BLKQ, 128), lambda i: (i, 0)),
        pl.BlockSpec((BLKQ, 128), lambda i: (i, 0)),
        pl.BlockSpec((1, 128), lambda i: (0, 0)),
        pl.BlockSpec((4, 128, 128), lambda i: (0, 0, 0)),
        pl.BlockSpec((1, 128), lambda i: (0, 0)),
        pl.BlockSpec((128, 128), lambda i: (0, 0)),
        pl.BlockSpec((1, 128), lambda i: (0, 0)),
    ],
    out_specs=pl.BlockSpec((BLKQ, 8), lambda i: (i, 0)),
    out_shape=jax.ShapeDtypeStruct((N_ACC // 4, 8), f32),
)


def _row32(n):
    return (((n >> 2) & 3) * (N_ACC // 16) + (n >> 4)) * 4 + (n & 3)


def kernel(x, edge_index, edge_attr, W1, b1, W2, b2, Wf1, bf1, Wf2, bf2):
    ei = edge_index.astype(i32)
    pad = jnp.full((E_PAD - E,), N, i32)
    src = jnp.concatenate([ei[0], pad])
    dst = jnp.concatenate([ei[1], pad])
    src_rows = src.reshape(ROWS, 128)
    dst_rows = dst.reshape(ROWS, 128)
    src32 = _row32(src)
    srcq_rows = jnp.stack([2 * src32, 2 * src32 + 1]).reshape(2, ROWS, 128)
    dstq_rows = _row32(dst).reshape(ROWS, 128)

    ea_pad = jnp.pad(edge_attr, ((0, N_ACC - N), (0, 3)))
    eap = ea_pad.reshape(N_ACC // 16, 128)
    W1p = jnp.pad(W1, ((0, 3), (0, 64)))
    b1p = jnp.pad(b1, (0, 64)).reshape(1, 128)
    W2p = jnp.pad(W2, ((0, 64), (0, 96)))
    b2rep = jnp.tile(b2, 4).reshape(1, 128)
    Wf1p = jnp.pad(Wf1, ((0, 96), (0, 112)))
    bf1p = jnp.pad(bf1, (0, 112)).reshape(1, 128)
    Wf2p = jnp.pad(Wf2, ((0, 112), (0, 126)))
    bf2p = jnp.pad(bf2, (0, 126)).reshape(1, 128)

    lanes = jnp.arange(128, dtype=i32)
    s8 = (lanes[:, None] == 8 * jnp.arange(16, dtype=i32)[None, :]) \
        .astype(f32)
    e8 = (lanes[None, :] // 8 == jnp.arange(16, dtype=i32)[:, None]) \
        .astype(f32)
    w1x = jnp.tile(W1p, (16, 1)).reshape(1, 128, 128) \
        * (lanes[None, :, None] // 8
           == jnp.arange(16, dtype=i32)[:, None, None]).astype(f32)
    wf1x = jnp.tile(jnp.pad(Wf1, ((0, 0), (0, 112))), (4, 1)) \
        .reshape(1, 128, 128) \
        * (lanes[None, :, None] // 32
           == jnp.arange(4, dtype=i32)[:, None, None]).astype(f32)

    zeros8 = jnp.zeros((N_ACC, 8), f32)
    zeros16 = jnp.zeros((N_ACC, 16), f32)
    ones8 = jnp.zeros((128, 8), f32).at[:, 0].set(1.0)

    degp = _deg(dst_rows, zeros8, ones8)
    dinv_r, y1p = _d1(degp.reshape(NC, N_ACC // 16, 128), eap, s8, e8)
    z1 = _p1(y1p.reshape(N_ACC, 8), src_rows, dst_rows, zeros8)
    y2q, dinvq = _d2(z1.reshape(NC, N_ACC // 16, 128), y1p, dinv_r, e8,
                     w1x, b1p, W2p)
    z2 = _p2(y2q.reshape(2 * N_ACC, 16), srcq_rows, dstq_rows, zeros16)
    oq = _d3(z2.reshape(N_ACC // 4, 128), y2q.reshape(N_ACC // 4, 128),
             dinvq.reshape(N_ACC // 4, 128), b2rep, wf1x, bf1p, Wf2p, bf2p)
    o = oq.reshape(4, N_ACC // 16, 4, 2).transpose(1, 0, 2, 3) \
        .reshape(N_ACC, 2)
    return o[:N]


# speedup vs baseline: 51.9985x; 1.0334x over previous
import functools

import jax
import jax.numpy as jnp
from jax import lax
from jax.experimental import pallas as pl
from jax.experimental.pallas import tpu as pltpu
from jax.experimental.pallas import tpu_sc as plsc

f32 = jnp.float32
i32 = jnp.int32

N = 100000
E = 1600000
NC, NS = 2, 16
N_ACC = 100352
ROWS = 12544
E_PAD = ROWS * 128
GR = 4
BLK = 1024
GRID = N_ACC // BLK

_MESH = dict(core_axis_name="c", subcore_axis_name="s", num_cores=NC,
             num_subcores=NS)
_SC_PARAMS = pltpu.CompilerParams(use_tc_tiling_on_sc=False)


def _make_deg():
    GR = 14
    rpw = ROWS // (NC * NS)
    ngroups = rpw // GR
    npairs = ngroups // 2
    srows = N_ACC // NS

    @functools.partial(
        pl.kernel,
        out_type=jax.ShapeDtypeStruct((NC, N_ACC, 8), f32),
        mesh=plsc.VectorSubcoreMesh(**_MESH),
        compiler_params=_SC_PARAMS,
        scratch_types=[
            pltpu.VMEM((GR, 128), i32),
            pltpu.VMEM((GR, 128), i32),
            pltpu.VMEM((128, 8), f32),
            pltpu.VMEM_SHARED((N_ACC, 8), f32),
            pltpu.SemaphoreType.DMA,
            pltpu.SemaphoreType.DMA,
        ],
    )
    def deg_kernel(dst_hbm, zeros_hbm, ones_hbm, out_hbm, idx0, idx1,
                   ones_v, acc_sh, isem0, isem1):
        c = lax.axis_index("c")
        s = lax.axis_index("s")
        w = c * NS + s
        base0 = w * rpw

        def iload(g, buf, sem):
            pltpu.async_copy(dst_hbm.at[pl.ds(base0 + g * GR, GR)], buf, sem)

        def iwait(buf, sem):
            pltpu.make_async_copy(dst_hbm.at[pl.ds(base0, GR)], buf,
                                  sem).wait()

        def scatters(buf):
            for j in range(GR):
                pltpu.sync_copy(ones_v, acc_sh.at[buf.at[j]], add=True)

        iload(0, idx0, isem0)
        iload(1, idx1, isem1)
        pltpu.sync_copy(ones_hbm, ones_v)
        pltpu.sync_copy(zeros_hbm.at[pl.ds(s * srows, srows)],
                        acc_sh.at[pl.ds(s * srows, srows)])
        plsc.subcore_barrier()

        def body(k, carry):
            g = 2 * k
            iwait(idx0, isem0)
            scatters(idx0)

            @pl.when(g + 2 < ngroups)
            def _():
                iload(g + 2, idx0, isem0)

            iwait(idx1, isem1)
            scatters(idx1)

            @pl.when(g + 3 < ngroups)
            def _():
                iload(g + 3, idx1, isem1)

            return carry

        lax.fori_loop(0, npairs, body, 0)
        plsc.subcore_barrier()
        pltpu.sync_copy(acc_sh.at[pl.ds(s * srows, srows)],
                        out_hbm.at[c, pl.ds(s * srows, srows)])

    return deg_kernel


def _make_agg(width, split_by_worker, GR):
    rpw = ROWS // (NC * NS) if split_by_worker else ROWS // NS
    ngroups = rpw // GR
    npairs = ngroups // 2
    srows = N_ACC // NS
    out_shape = ((NC, N_ACC, width) if split_by_worker
                 else (N_ACC, NC, width))

    @functools.partial(
        pl.kernel,
        out_type=jax.ShapeDtypeStruct(out_shape, f32),
        mesh=plsc.VectorSubcoreMesh(**_MESH),
        compiler_params=_SC_PARAMS,
        scratch_types=[
            pltpu.VMEM((GR, 128), i32),
            pltpu.VMEM((GR, 128), i32),
            pltpu.VMEM((GR, 128), i32),
            pltpu.VMEM((GR, 128), i32),
            pltpu.VMEM((GR * 128, width), f32),
            pltpu.VMEM((GR * 128, width), f32),
            pltpu.VMEM_SHARED((N_ACC, width), f32),
            pltpu.SemaphoreType.DMA,
            pltpu.SemaphoreType.DMA,
            pltpu.SemaphoreType.DMA,
            pltpu.SemaphoreType.DMA,
        ],
    )
    def agg_kernel(table_hbm, src_hbm, dst_hbm, zeros_hbm, out_hbm,
                   sidx0, sidx1, didx0, didx1, rows0, rows1, acc_sh,
                   isem0, isem1, gsem0, gsem1):
        c = lax.axis_index("c")
        s = lax.axis_index("s")
        if split_by_worker:
            base0 = (c * NS + s) * rpw
            tbl = table_hbm
            sref = src_hbm
        else:
            base0 = s * rpw
            tbl = table_hbm
            sref = src_hbm.at[c]

        def sload(g, sbuf, sem):
            pltpu.async_copy(sref.at[pl.ds(base0 + g * GR, GR)], sbuf, sem)

        def dload(g, dbuf, sem):
            pltpu.async_copy(dst_hbm.at[pl.ds(base0 + g * GR, GR)], dbuf, sem)

        def ibwait(buf, sem):
            pltpu.make_async_copy(dst_hbm.at[pl.ds(base0, GR)], buf,
                                  sem).wait()

        def gathers(sbuf, rbuf, sem):
            for j in range(GR):
                pltpu.async_copy(tbl.at[sbuf.at[j]],
                                 rbuf.at[pl.ds(j * 128, 128)], sem)

        def gwait(rbuf, sem):
            for j in range(GR):
                pltpu.make_async_copy(tbl.at[pl.ds(0, 128)],
                                      rbuf.at[pl.ds(j * 128, 128)],
                                      sem).wait()

        def scatters(rbuf, dbuf):
            for j in range(GR):
                pltpu.sync_copy(rbuf.at[pl.ds(j * 128, 128)],
                                acc_sh.at[dbuf.at[j]], add=True)

        sload(0, sidx0, isem0)
        dload(0, didx0, isem0)
        sload(1, sidx1, isem1)
        dload(1, didx1, isem1)
        ibwait(sidx0, isem0)
        ibwait(didx0, isem0)
        gathers(sidx0, rows0, gsem0)
        pltpu.sync_copy(zeros_hbm.at[pl.ds(s * srows, srows)],
                        acc_sh.at[pl.ds(s * srows, srows)])
        plsc.subcore_barrier()

        def body(k, carry):
            g = 2 * k
            ibwait(sidx1, isem1)
            ibwait(didx1, isem1)
            gathers(sidx1, rows1, gsem1)
            gwait(rows0, gsem0)

            @pl.when(g + 2 < ngroups)
            def _():
                sload(g + 2, sidx0, isem0)

            scatters(rows0, didx0)

            @pl.when(g + 2 < ngroups)
            def _():
                dload(g + 2, didx0, isem0)
                ibwait(sidx0, isem0)
                ibwait(didx0, isem0)
                gathers(sidx0, rows0, gsem0)

            gwait(rows1, gsem1)

            @pl.when(g + 3 < ngroups)
            def _():
                sload(g + 3, sidx1, isem1)

            scatters(rows1, didx1)

            @pl.when(g + 3 < ngroups)
            def _():
                dload(g + 3, didx1, isem1)

            return carry

        lax.fori_loop(0, npairs, body, 0)
        plsc.subcore_barrier()
        if split_by_worker:
            pltpu.sync_copy(acc_sh.at[pl.ds(s * srows, srows)],
                            out_hbm.at[c, pl.ds(s * srows, srows)])
        else:
            pltpu.sync_copy(acc_sh.at[pl.ds(s * srows, srows)],
                            out_hbm.at[pl.ds(s * srows, srows), c])

    return agg_kernel


_deg = _make_deg()
_p1 = _make_agg(8, True, 14)
_p2 = _make_agg(16, False, 4)

BLKP = 128
GRIDP = (N_ACC // 16) // BLKP
BLKQ = 512
GRIDQ = (N_ACC // 4) // BLKQ


def _d1_body(degp_ref, eap_ref, s8_ref, e8_ref, dinv_ref, y1p_ref):
    degsum = degp_ref[0] + degp_ref[1]
    deg = jnp.dot(degsum, s8_ref[...], preferred_element_type=f32) + 1.0
    dinv = 1.0 / jnp.sqrt(deg)
    dinv_ref[...] = dinv
    y1p_ref[...] = eap_ref[...] * jnp.dot(dinv, e8_ref[...],
                                          preferred_element_type=f32)


_d1 = pl.pallas_call(
    _d1_body,
    grid=(GRIDP,),
    in_specs=[
        pl.BlockSpec((NC, BLKP, 128), lambda i: (0, i, 0)),
        pl.BlockSpec((BLKP, 128), lambda i: (i, 0)),
        pl.BlockSpec((128, 16), lambda i: (0, 0)),
        pl.BlockSpec((16, 128), lambda i: (0, 0)),
    ],
    out_specs=[
        pl.BlockSpec((BLKP, 16), lambda i: (i, 0)),
        pl.BlockSpec((BLKP, 128), lambda i: (i, 0)),
    ],
    out_shape=[
        jax.ShapeDtypeStruct((N_ACC // 16, 16), f32),
        jax.ShapeDtypeStruct((N_ACC // 16, 128), f32),
    ],
)


def _d2_body(z1p_ref, y1p_ref, dinv_ref, e8_ref, w1x_ref, b1_ref, w2_ref,
             y2q_ref, dinvq_ref):
    dinv = dinv_ref[...]
    agg1p = (z1p_ref[0] + z1p_ref[1] + y1p_ref[...]) * jnp.dot(
        dinv, e8_ref[...], preferred_element_type=f32)
    for q in range(4):
        parts = []
        dparts = []
        for j in range(4):
            i = 4 * q + j
            h = jnp.maximum(
                jnp.dot(agg1p, w1x_ref[i], preferred_element_type=f32)
                + b1_ref[...], 0.0)
            t = jnp.dot(h, w2_ref[...], preferred_element_type=f32)
            db = dinv[:, i:i + 1]
            parts.append((t * db)[:, :32])
            dparts.append(jnp.broadcast_to(db, (BLKP, 32)))
        y2q_ref[q] = jnp.concatenate(parts, axis=1)
        dinvq_ref[q] = jnp.concatenate(dparts, axis=1)


_d2 = pl.pallas_call(
    _d2_body,
    grid=(GRIDP,),
    in_specs=[
        pl.BlockSpec((NC, BLKP, 128), lambda i: (0, i, 0)),
        pl.BlockSpec((BLKP, 128), lambda i: (i, 0)),
        pl.BlockSpec((BLKP, 16), lambda i: (i, 0)),
        pl.BlockSpec((16, 128), lambda i: (0, 0)),
        pl.BlockSpec((16, 128, 128), lambda i: (0, 0, 0)),
        pl.BlockSpec((1, 128), lambda i: (0, 0)),
        pl.BlockSpec((128, 128), lambda i: (0, 0)),
    ],
    out_specs=[
        pl.BlockSpec((4, BLKP, 128), lambda i: (0, i, 0)),
        pl.BlockSpec((4, BLKP, 128), lambda i: (0, i, 0)),
    ],
    out_shape=[
        jax.ShapeDtypeStruct((4, N_ACC // 16, 128), f32),
        jax.ShapeDtypeStruct((4, N_ACC // 16, 128), f32),
    ],
)


def _d3_body(z2p_ref, y2q_ref, dinvq_ref, b2_ref, wf1x_ref, bf1_ref,
             wf2_ref, bf2_ref, oq_ref):
    u = dinvq_ref[...] * (z2p_ref[...] + y2q_ref[...])
    h2 = jnp.maximum(u + b2_ref[...], 0.0)
    parts = []
    for j in range(4):
        h3 = jnp.maximum(
            jnp.dot(h2, wf1x_ref[j], preferred_element_type=f32)
            + bf1_ref[...], 0.0)
        o = jnp.dot(h3, wf2_ref[...], preferred_element_type=f32) \
            + bf2_ref[...]
        parts.append(o[:, :2])
    oq_ref[...] = jnp.concatenate(parts, axis=1)


_d3 = pl.pallas_call(
    _d3_body,
    grid=(GRIDQ,),
    in_specs=[
        pl.BlockSpec((BLKQ, 128), lambda i: (i, 0)),
        pl.BlockSpec((BLKQ, 128), lambda i: (i, 0)),
        pl.BlockSpec((BLKQ, 128), lambda i: (i, 0)),
        pl.BlockSpec((1, 128), lambda i: (0, 0)),
        pl.BlockSpec((4, 128, 128), lambda i: (0, 0, 0)),
        pl.BlockSpec((1, 128), lambda i: (0, 0)),
        pl.BlockSpec((128, 128), lambda i: (0, 0)),
        pl.BlockSpec((1, 128), lambda i: (0, 0)),
    ],
    out_specs=pl.BlockSpec((BLKQ, 8), lambda i: (i, 0)),
    out_shape=jax.ShapeDtypeStruct((N_ACC // 4, 8), f32),
)


def _row32(n):
    return (((n >> 2) & 3) * (N_ACC // 16) + (n >> 4)) * 4 + (n & 3)


def kernel(x, edge_index, edge_attr, W1, b1, W2, b2, Wf1, bf1, Wf2, bf2):
    ei = edge_index.astype(i32)
    pad = jnp.full((E_PAD - E,), N, i32)
    src = jnp.concatenate([ei[0], pad])
    dst = jnp.concatenate([ei[1], pad])
    src_rows = src.reshape(ROWS, 128)
    dst_rows = dst.reshape(ROWS, 128)
    src32 = _row32(src)
    srcq_rows = jnp.stack([2 * src32, 2 * src32 + 1]).reshape(2, ROWS, 128)
    dstq_rows = _row32(dst).reshape(ROWS, 128)

    ea_pad = jnp.pad(edge_attr, ((0, N_ACC - N), (0, 3)))
    eap = ea_pad.reshape(N_ACC // 16, 128)
    W1p = jnp.pad(W1, ((0, 3), (0, 64)))
    b1p = jnp.pad(b1, (0, 64)).reshape(1, 128)
    W2p = jnp.pad(W2, ((0, 64), (0, 96)))
    b2rep = jnp.tile(b2, 4).reshape(1, 128)
    Wf1p = jnp.pad(Wf1, ((0, 96), (0, 112)))
    bf1p = jnp.pad(bf1, (0, 112)).reshape(1, 128)
    Wf2p = jnp.pad(Wf2, ((0, 112), (0, 126)))
    bf2p = jnp.pad(bf2, (0, 126)).reshape(1, 128)

    lanes = jnp.arange(128, dtype=i32)
    s8 = (lanes[:, None] == 8 * jnp.arange(16, dtype=i32)[None, :]) \
        .astype(f32)
    e8 = (lanes[None, :] // 8 == jnp.arange(16, dtype=i32)[:, None]) \
        .astype(f32)
    w1x = jnp.tile(W1p, (16, 1)).reshape(1, 128, 128) \
        * (lanes[None, :, None] // 8
           == jnp.arange(16, dtype=i32)[:, None, None]).astype(f32)
    wf1x = jnp.tile(jnp.pad(Wf1, ((0, 0), (0, 112))), (4, 1)) \
        .reshape(1, 128, 128) \
        * (lanes[None, :, None] // 32
           == jnp.arange(4, dtype=i32)[:, None, None]).astype(f32)

    zeros8 = jnp.zeros((N_ACC, 8), f32)
    zeros16 = jnp.zeros((N_ACC, 16), f32)
    ones8 = jnp.zeros((128, 8), f32).at[:, 0].set(1.0)

    degp = _deg(dst_rows, zeros8, ones8)
    dinv_r, y1p = _d1(degp.reshape(NC, N_ACC // 16, 128), eap, s8, e8)
    z1 = _p1(y1p.reshape(N_ACC, 8), src_rows, dst_rows, zeros8)
    y2q, dinvq = _d2(z1.reshape(NC, N_ACC // 16, 128), y1p, dinv_r, e8,
                     w1x, b1p, W2p)
    z2 = _p2(y2q.reshape(2 * N_ACC, 16), srcq_rows, dstq_rows, zeros16)
    oq = _d3(z2.reshape(N_ACC // 4, 128), y2q.reshape(N_ACC // 4, 128),
             dinvq.reshape(N_ACC // 4, 128), b2rep, wf1x, bf1p, Wf2p, bf2p)
    o = oq.reshape(4, N_ACC // 16, 4, 2).transpose(1, 0, 2, 3) \
        .reshape(N_ACC, 2)
    return o[:N]
